# baseline (speedup 1.0000x reference)
import functools

import jax
import jax.numpy as jnp
from jax import lax
from jax.experimental import pallas as pl
from jax.experimental.pallas import tpu as pltpu
from jax.experimental.pallas import tpu_sc as plsc

N_NODES = 10000
N_EDGES = 160000
DIM = 256
HID = 32

NC = 2
NS = 16
NW = NC * NS
CHUNK = 128
CPW = 40
EPW = CHUNK * CPW
EPAD = NW * EPW
NPAD = 10240
PW = 48
ZROWS = NPAD // NS
RING = 4

BN = 1280
BE = 1280
VALID_BLOCKS = N_EDGES // BE

_MESH = plsc.VectorSubcoreMesh(
    core_axis_name="c", subcore_axis_name="s", num_cores=NC, num_subcores=NS)

_DOT = functools.partial(
    jnp.dot, preferred_element_type=jnp.float32, precision=lax.Precision.HIGHEST)


def _pq_body(x_ref, w_ref, p1_ref, q_ref):
    pq = _DOT(x_ref[...], w_ref[...])
    ones = jnp.ones((BN, 1), jnp.float32)
    zeros = jnp.zeros((BN, PW - HID - 1), jnp.float32)
    p1_ref[...] = jnp.concatenate([pq[:, :HID], ones, zeros], axis=1)
    q_ref[...] = pq[:, HID:]


def _pq_tables(x_pad, w_cat):
    return pl.pallas_call(
        _pq_body,
        grid=(NPAD // BN,),
        in_specs=[
            pl.BlockSpec((BN, DIM), lambda i: (i, 0)),
            pl.BlockSpec((DIM, 2 * HID), lambda i: (0, 0)),
        ],
        out_specs=[
            pl.BlockSpec((BN, PW), lambda i: (i, 0)),
            pl.BlockSpec((BN, HID), lambda i: (i, 0)),
        ],
        out_shape=[
            jax.ShapeDtypeStruct((NPAD, PW), jnp.float32),
            jax.ShapeDtypeStruct((NPAD, HID), jnp.float32),
        ],
    )(x_pad, w_cat)


def _sc_seg_p_body(src_hbm, dst_hbm, p1_hbm, zero_hbm, out_hbm,
                   src_v, dst_v, rows_v, zbuf_v, acc_s, sg, ss):
    c = lax.axis_index("c")
    s = lax.axis_index("s")
    wid = s * NC + c
    r0 = s * ZROWS

    pltpu.sync_copy(zero_hbm, zbuf_v)
    pltpu.sync_copy(zbuf_v, acc_s.at[pl.ds(r0, ZROWS)])
    pltpu.sync_copy(src_hbm.at[wid], src_v)
    pltpu.sync_copy(dst_hbm.at[wid], dst_v)
    plsc.subcore_barrier()

    def group(g, carry):
        j0 = g * RING
        for t in range(RING):
            pltpu.async_copy(p1_hbm.at[src_v.at[j0 + t]], rows_v.at[t],
                             sg.at[t])
        for t in range(RING):
            pltpu.make_async_copy(p1_hbm.at[src_v.at[j0 + t]], rows_v.at[t],
                                  sg.at[t]).wait()
            pltpu.async_copy(rows_v.at[t], acc_s.at[dst_v.at[j0 + t]],
                             ss.at[t], add=True)
        for t in range(RING):
            pltpu.make_async_copy(rows_v.at[t], acc_s.at[dst_v.at[j0 + t]],
                                  ss.at[t]).wait()
        return carry

    lax.fori_loop(0, CPW // RING, group, 0)
    plsc.subcore_barrier()
    pltpu.sync_copy(acc_s.at[pl.ds(r0, ZROWS)], zbuf_v)
    pltpu.sync_copy(zbuf_v, out_hbm.at[c, pl.ds(r0, ZROWS)])


def _sc_seg_p(src3, dst3, p1, zero_rows):
    return pl.kernel(
        _sc_seg_p_body,
        out_type=jax.ShapeDtypeStruct((NC, NPAD, PW), jnp.float32),
        mesh=_MESH,
        compiler_params=pltpu.CompilerParams(use_tc_tiling_on_sc=False),
        scratch_types=[
            pltpu.VMEM((CPW, CHUNK), jnp.int32),
            pltpu.VMEM((CPW, CHUNK), jnp.int32),
            pltpu.VMEM((RING, CHUNK, PW), jnp.float32),
            pltpu.VMEM((ZROWS, PW), jnp.float32),
            pltpu.VMEM_SHARED((NPAD, PW), jnp.float32),
            pltpu.SemaphoreType.DMA((RING,)),
            pltpu.SemaphoreType.DMA((RING,)),
        ],
    )(src3, dst3, p1, zero_rows)


def _node1_body(spd_ref, p1_ref, q_ref, wab_ref, bn1_ref, be1_ref,
                x1_ref, g1_ref, g2_ref):
    sfull = spd_ref[0] + spd_ref[1]
    deg = sfull[:, HID:HID + 1]
    denom = jnp.maximum(deg, 1.0)
    p = p1_ref[:, :HID]
    x1 = jnp.maximum((deg * p - sfull[:, :HID]) / denom + bn1_ref[0, :], 0.0)
    g12 = _DOT(x1, wab_ref[...])
    qv = q_ref[...]
    x1_ref[...] = x1
    g1_ref[...] = g12[:, :HID] - qv
    g2_ref[...] = g12[:, HID:] + qv + be1_ref[0, :]


def _node1(spd, p1, q, wab, bn1, be1):
    return pl.pallas_call(
        _node1_body,
        grid=(NPAD // BN,),
        in_specs=[
            pl.BlockSpec((NC, BN, PW), lambda i: (0, i, 0)),
            pl.BlockSpec((BN, PW), lambda i: (i, 0)),
            pl.BlockSpec((BN, HID), lambda i: (i, 0)),
            pl.BlockSpec((HID, 2 * HID), lambda i: (0, 0)),
            pl.BlockSpec((1, HID), lambda i: (0, 0)),
            pl.BlockSpec((1, HID), lambda i: (0, 0)),
        ],
        out_specs=[
            pl.BlockSpec((BN, HID), lambda i: (i, 0)),
            pl.BlockSpec((BN, HID), lambda i: (i, 0)),
            pl.BlockSpec((BN, HID), lambda i: (i, 0)),
        ],
        out_shape=[
            jax.ShapeDtypeStruct((NPAD, HID), jnp.float32),
            jax.ShapeDtypeStruct((NPAD, HID), jnp.float32),
            jax.ShapeDtypeStruct((NPAD, HID), jnp.float32),
        ],
    )(spd, p1, q, wab, bn1, be1)


def _relu_rows(e_ref):
    def rows(i, carry):
        r = i * 4
        for t in range(4):
            for h in (0, 16):
                sl = pl.ds(h, 16)
                e_ref[r + t, sl] = jnp.maximum(e_ref[r + t, sl], 0.0)
        return carry

    lax.fori_loop(0, CHUNK // 4, rows, 0)


def _sc_edge1_body(src_hbm, dst_hbm, g1_hbm, g2_hbm, zero_hbm,
                   e1_hbm, out_hbm,
                   src_v, dst_v, e_v, zbuf_v, acc_s, sga, sgb, sst, ssc):
    c = lax.axis_index("c")
    s = lax.axis_index("s")
    wid = s * NC + c
    r0 = s * ZROWS
    base = wid * EPW

    pltpu.sync_copy(zero_hbm, zbuf_v)
    pltpu.sync_copy(zbuf_v, acc_s.at[pl.ds(r0, ZROWS)])
    pltpu.sync_copy(src_hbm.at[wid], src_v)
    pltpu.sync_copy(dst_hbm.at[wid], dst_v)
    plsc.subcore_barrier()

    def group(g, carry):
        j0 = g * RING
        for t in range(RING):
            pltpu.async_copy(g1_hbm.at[src_v.at[j0 + t]], e_v.at[t],
                             sga.at[t])
        for t in range(RING):
            pltpu.make_async_copy(g1_hbm.at[src_v.at[j0 + t]], e_v.at[t],
                                  sga.at[t]).wait()
            pltpu.async_copy(g2_hbm.at[dst_v.at[j0 + t]], e_v.at[t],
                             sgb.at[t], add=True)
        for t in range(RING):
            pltpu.make_async_copy(g2_hbm.at[dst_v.at[j0 + t]], e_v.at[t],
                                  sgb.at[t]).wait()
            _relu_rows(e_v.at[t])
            pltpu.async_copy(e_v.at[t],
                             e1_hbm.at[pl.ds(base + (j0 + t) * CHUNK, CHUNK)],
                             sst.at[t])
            pltpu.async_copy(e_v.at[t], acc_s.at[dst_v.at[j0 + t]],
                             ssc.at[t], add=True)
        for t in range(RING):
            pltpu.make_async_copy(
                e_v.at[t], e1_hbm.at[pl.ds(base + (j0 + t) * CHUNK, CHUNK)],
                sst.at[t]).wait()
            pltpu.make_async_copy(e_v.at[t], acc_s.at[dst_v.at[j0 + t]],
                                  ssc.at[t]).wait()
        return carry

    lax.fori_loop(0, CPW // RING, group, 0)
    plsc.subcore_barrier()
    pltpu.sync_copy(acc_s.at[pl.ds(r0, ZROWS)], zbuf_v)
    pltpu.sync_copy(zbuf_v, out_hbm.at[c, pl.ds(r0, ZROWS)])


def _sc_edge1(src3, dst3, g1, g2, zero_rows):
    return pl.kernel(
        _sc_edge1_body,
        out_type=[
            jax.ShapeDtypeStruct((EPAD, HID), jnp.float32),
            jax.ShapeDtypeStruct((NC, NPAD, HID), jnp.float32),
        ],
        mesh=_MESH,
        compiler_params=pltpu.CompilerParams(use_tc_tiling_on_sc=False),
        scratch_types=[
            pltpu.VMEM((CPW, CHUNK), jnp.int32),
            pltpu.VMEM((CPW, CHUNK), jnp.int32),
            pltpu.VMEM((RING, CHUNK, HID), jnp.float32),
            pltpu.VMEM((ZROWS, HID), jnp.float32),
            pltpu.VMEM_SHARED((NPAD, HID), jnp.float32),
            pltpu.SemaphoreType.DMA((RING,)),
            pltpu.SemaphoreType.DMA((RING,)),
            pltpu.SemaphoreType.DMA((RING,)),
            pltpu.SemaphoreType.DMA((RING,)),
        ],
    )(src3, dst3, g1, g2, zero_rows)


def _node2_body(sed_ref, spd_ref, x1_ref, wn2_ref, wab2_ref, bn2_ref, be2_ref,
                g3_ref, g4_ref):
    se = sed_ref[0] + sed_ref[1]
    deg = spd_ref[0][:, HID:HID + 1] + spd_ref[1][:, HID:HID + 1]
    denom = jnp.maximum(deg, 1.0)
    agg2 = se / denom
    cat = jnp.concatenate([x1_ref[...], agg2], axis=1)
    x2 = jnp.maximum(_DOT(cat, wn2_ref[...]) + bn2_ref[0, :], 0.0)
    g34 = _DOT(x2, wab2_ref[...])
    g3_ref[...] = g34[:, :HID]
    g4_ref[...] = g34[:, HID:] + be2_ref[0, :]


def _node2(sed, spd, x1, wn2, wab2, bn2, be2):
    return pl.pallas_call(
        _node2_body,
        grid=(NPAD // BN,),
        in_specs=[
            pl.BlockSpec((NC, BN, HID), lambda i: (0, i, 0)),
            pl.BlockSpec((NC, BN, PW), lambda i: (0, i, 0)),
            pl.BlockSpec((BN, HID), lambda i: (i, 0)),
            pl.BlockSpec((2 * HID, HID), lambda i: (0, 0)),
            pl.BlockSpec((HID, 2 * HID), lambda i: (0, 0)),
            pl.BlockSpec((1, HID), lambda i: (0, 0)),
            pl.BlockSpec((1, HID), lambda i: (0, 0)),
        ],
        out_specs=[
            pl.BlockSpec((BN, HID), lambda i: (i, 0)),
            pl.BlockSpec((BN, HID), lambda i: (i, 0)),
        ],
        out_shape=[
            jax.ShapeDtypeStruct((NPAD, HID), jnp.float32),
            jax.ShapeDtypeStruct((NPAD, HID), jnp.float32),
        ],
    )(sed, spd, x1, wn2, wab2, bn2, be2)


def _sc_edge2_body(src_hbm, dst_hbm, g3_hbm, g4_hbm, z2_hbm,
                   src_v, dst_v, e_v, sga, sgb, sst):
    c = lax.axis_index("c")
    s = lax.axis_index("s")
    wid = s * NC + c
    base = wid * EPW

    pltpu.sync_copy(src_hbm.at[wid], src_v)
    pltpu.sync_copy(dst_hbm.at[wid], dst_v)

    def group(g, carry):
        j0 = g * RING
        for t in range(RING):
            pltpu.async_copy(g3_hbm.at[src_v.at[j0 + t]], e_v.at[t],
                             sga.at[t])
        for t in range(RING):
            pltpu.make_async_copy(g3_hbm.at[src_v.at[j0 + t]], e_v.at[t],
                                  sga.at[t]).wait()
            pltpu.async_copy(g4_hbm.at[dst_v.at[j0 + t]], e_v.at[t],
                             sgb.at[t], add=True)
        for t in range(RING):
            pltpu.make_async_copy(g4_hbm.at[dst_v.at[j0 + t]], e_v.at[t],
                                  sgb.at[t]).wait()
            pltpu.async_copy(e_v.at[t],
                             z2_hbm.at[pl.ds(base + (j0 + t) * CHUNK, CHUNK)],
                             sst.at[t])
        for t in range(RING):
            pltpu.make_async_copy(
                e_v.at[t], z2_hbm.at[pl.ds(base + (j0 + t) * CHUNK, CHUNK)],
                sst.at[t]).wait()
        return carry

    lax.fori_loop(0, CPW // RING, group, 0)


def _sc_edge2(src3, dst3, g3, g4):
    return pl.kernel(
        _sc_edge2_body,
        out_type=jax.ShapeDtypeStruct((EPAD, HID), jnp.float32),
        mesh=_MESH,
        compiler_params=pltpu.CompilerParams(use_tc_tiling_on_sc=False),
        scratch_types=[
            pltpu.VMEM((CPW, CHUNK), jnp.int32),
            pltpu.VMEM((CPW, CHUNK), jnp.int32),
            pltpu.VMEM((RING, CHUNK, HID), jnp.float32),
            pltpu.SemaphoreType.DMA((RING,)),
            pltpu.SemaphoreType.DMA((RING,)),
            pltpu.SemaphoreType.DMA((RING,)),
        ],
    )(src3, dst3, g3, g4)


def _final_body(z2_ref, e1_ref, c2_ref, wc_ref, bc_ref, et_ref,
                pred_ref, loss_ref, s_ref):
    i = pl.program_id(0)
    t = _DOT(e1_ref[...], c2_ref[...])
    e2 = jnp.maximum(z2_ref[...] + t, 0.0)
    logit = _DOT(e2, wc_ref[...]) + bc_ref[0, 0]
    pred = 1.0 / (1.0 + jnp.exp(-logit))
    pred_ref[...] = pred

    @pl.when(i == 0)
    def _():
        for k in range(5):
            s_ref[k] = 0.0

    @pl.when(i < VALID_BLOCKS)
    def _():
        p2 = pred * pred
        pc = jnp.clip(pred, 1e-7, 1.0 - 1e-7)
        y = et_ref[...]
        bce = -(y * jnp.log(pc) + (1.0 - y) * jnp.log(1.0 - pc))
        s_ref[0] += jnp.sum(pred)
        s_ref[1] += jnp.sum(p2)
        s_ref[2] += jnp.sum(p2 * pred)
        s_ref[3] += jnp.sum(p2 * p2)
        s_ref[4] += jnp.sum(bce)

    @pl.when(i == pl.num_programs(0) - 1)
    def _():
        e = float(N_EDGES)
        m1 = s_ref[0] / e
        m2 = s_ref[1] / e
        m3 = s_ref[2] / e
        m4 = s_ref[3] / e
        msq = m1 * m1
        cm4 = m4 - 4.0 * m1 * m3 + 6.0 * msq * m2 - 3.0 * msq * msq
        cm4 = jnp.maximum(cm4, 0.0)
        reg = -jnp.sqrt(jnp.sqrt(cm4))
        loss_ref[0, 0] = s_ref[4] / e + 0.1 * reg


def _final(z2, e1, c2, wc, bc, et):
    return pl.pallas_call(
        _final_body,
        grid=(EPAD // BE,),
        in_specs=[
            pl.BlockSpec((BE, HID), lambda i: (i, 0)),
            pl.BlockSpec((BE, HID), lambda i: (i, 0)),
            pl.BlockSpec((HID, HID), lambda i: (0, 0)),
            pl.BlockSpec((HID, 1), lambda i: (0, 0)),
            pl.BlockSpec((1, 1), lambda i: (0, 0), memory_space=pltpu.SMEM),
            pl.BlockSpec((BE, 1), lambda i: (i, 0)),
        ],
        out_specs=[
            pl.BlockSpec((BE, 1), lambda i: (i, 0)),
            pl.BlockSpec((1, 1), lambda i: (0, 0), memory_space=pltpu.SMEM),
        ],
        out_shape=[
            jax.ShapeDtypeStruct((EPAD, 1), jnp.float32),
            jax.ShapeDtypeStruct((1, 1), jnp.float32),
        ],
        scratch_shapes=[pltpu.SMEM((8,), jnp.float32)],
    )(z2, e1, c2, wc, bc, et)


def kernel(x, edge_index, edge_type,
           W_n1, b_n1, W_e1, b_e1, W_n2, b_n2, W_e2, b_e2, W_c, b_c):
    f32 = jnp.float32
    src3 = jnp.pad(edge_index[0], (0, EPAD - N_EDGES),
                   constant_values=N_NODES).reshape(NW, CPW, CHUNK)
    dst3 = jnp.pad(edge_index[1], (0, EPAD - N_EDGES),
                   constant_values=N_NODES).reshape(NW, CPW, CHUNK)
    x_pad = jnp.pad(x, ((0, NPAD - N_NODES), (0, 0)))
    w_cat = jnp.concatenate([W_n1[DIM:], W_e1[2 * HID:]], axis=1)

    p1, q = _pq_tables(x_pad, w_cat)

    zero_p = jnp.zeros((ZROWS, PW), f32)
    spd = _sc_seg_p(src3, dst3, p1, zero_p)

    wab = jnp.concatenate([W_e1[:HID], W_e1[HID:2 * HID]], axis=1)
    x1, g1, g2 = _node1(spd, p1, q, wab,
                        b_n1.reshape(1, HID), b_e1.reshape(1, HID))

    zero_h = jnp.zeros((ZROWS, HID), f32)
    e1, sed = _sc_edge1(src3, dst3, g1, g2, zero_h)

    wab2 = jnp.concatenate([W_e2[:HID], W_e2[HID:2 * HID]], axis=1)
    g3, g4 = _node2(sed, spd, x1, W_n2, wab2,
                    b_n2.reshape(1, HID), b_e2.reshape(1, HID))

    z2 = _sc_edge2(src3, dst3, g3, g4)

    et = jnp.pad(edge_type.astype(f32), (0, EPAD - N_EDGES)).reshape(EPAD, 1)
    pred_pad, loss = _final(z2, e1, W_e2[2 * HID:], W_c,
                            b_c.reshape(1, 1), et)
    return pred_pad[:N_EDGES, 0], loss[0, 0]


# speedup vs baseline: 1.0850x; 1.0850x over previous
import functools

import jax
import jax.numpy as jnp
from jax import lax
from jax.experimental import pallas as pl
from jax.experimental.pallas import tpu as pltpu
from jax.experimental.pallas import tpu_sc as plsc

N_NODES = 10000
N_EDGES = 160000
DIM = 256
HID = 32

NC = 2
NS = 16
CHUNK = 128
TOTC = 1280
EPAD = TOTC * CHUNK
NPAD = 10240
PW = 48
ZROWS = NPAD // NS
RING = 4

FAST_CORE = 1
C_FAST = 60
C_SLOW = 20
IDXROWS = C_FAST
TOTC_PAD = TOTC + IDXROWS
EPAD_IDX = TOTC_PAD * CHUNK

BN = 1280
BE = 1280
VALID_BLOCKS = N_EDGES // BE

_MESH = plsc.VectorSubcoreMesh(
    core_axis_name="c", subcore_axis_name="s", num_cores=NC, num_subcores=NS)

_DOT = functools.partial(
    jnp.dot, preferred_element_type=jnp.float32, precision=lax.Precision.HIGHEST)


def _my_chunks(c, s):
    is_fast = c == FAST_CORE
    nmy = jnp.where(is_fast, C_FAST, C_SLOW)
    cbase = jnp.where(is_fast, s * C_FAST, NS * C_FAST + s * C_SLOW)
    return nmy, cbase


def _pq_body(x_ref, w_ref, p1_ref, q_ref):
    pq = _DOT(x_ref[...], w_ref[...])
    ones = jnp.ones((BN, 1), jnp.float32)
    zeros = jnp.zeros((BN, PW - HID - 1), jnp.float32)
    p1_ref[...] = jnp.concatenate([pq[:, :HID], ones, zeros], axis=1)
    q_ref[...] = pq[:, HID:]


def _pq_tables(x_pad, w_cat):
    return pl.pallas_call(
        _pq_body,
        grid=(NPAD // BN,),
        in_specs=[
            pl.BlockSpec((BN, DIM), lambda i: (i, 0)),
            pl.BlockSpec((DIM, 2 * HID), lambda i: (0, 0)),
        ],
        out_specs=[
            pl.BlockSpec((BN, PW), lambda i: (i, 0)),
            pl.BlockSpec((BN, HID), lambda i: (i, 0)),
        ],
        out_shape=[
            jax.ShapeDtypeStruct((NPAD, PW), jnp.float32),
            jax.ShapeDtypeStruct((NPAD, HID), jnp.float32),
        ],
    )(x_pad, w_cat)


def _sc_seg_p_body(src_hbm, dst_hbm, p1_hbm, zero_hbm, out_hbm,
                   src_v, dst_v, rows_v, zbuf_v, acc_s, sg, ss):
    c = lax.axis_index("c")
    s = lax.axis_index("s")
    r0 = s * ZROWS
    nmy, cbase = _my_chunks(c, s)

    pltpu.sync_copy(zero_hbm, zbuf_v)
    pltpu.sync_copy(zbuf_v, acc_s.at[pl.ds(r0, ZROWS)])
    pltpu.sync_copy(src_hbm.at[pl.ds(cbase, IDXROWS)], src_v)
    pltpu.sync_copy(dst_hbm.at[pl.ds(cbase, IDXROWS)], dst_v)
    plsc.subcore_barrier()

    def group(g, carry):
        j0 = g * RING
        for t in range(RING):
            pltpu.async_copy(p1_hbm.at[src_v.at[j0 + t]], rows_v.at[t],
                             sg.at[t])
        for t in range(RING):
            pltpu.make_async_copy(p1_hbm.at[src_v.at[j0 + t]], rows_v.at[t],
                                  sg.at[t]).wait()
            pltpu.async_copy(rows_v.at[t], acc_s.at[dst_v.at[j0 + t]],
                             ss.at[t], add=True)
        for t in range(RING):
            pltpu.make_async_copy(rows_v.at[t], acc_s.at[dst_v.at[j0 + t]],
                                  ss.at[t]).wait()
        return carry

    lax.fori_loop(0, nmy // RING, group, 0)
    plsc.subcore_barrier()
    pltpu.sync_copy(acc_s.at[pl.ds(r0, ZROWS)], zbuf_v)
    pltpu.sync_copy(zbuf_v, out_hbm.at[c, pl.ds(r0, ZROWS)])


def _sc_seg_p(src3, dst3, p1, zero_rows):
    return pl.kernel(
        _sc_seg_p_body,
        out_type=jax.ShapeDtypeStruct((NC, NPAD, PW), jnp.float32),
        mesh=_MESH,
        compiler_params=pltpu.CompilerParams(use_tc_tiling_on_sc=False),
        scratch_types=[
            pltpu.VMEM((IDXROWS, CHUNK), jnp.int32),
            pltpu.VMEM((IDXROWS, CHUNK), jnp.int32),
            pltpu.VMEM((RING, CHUNK, PW), jnp.float32),
            pltpu.VMEM((ZROWS, PW), jnp.float32),
            pltpu.VMEM_SHARED((NPAD, PW), jnp.float32),
            pltpu.SemaphoreType.DMA((RING,)),
            pltpu.SemaphoreType.DMA((RING,)),
        ],
    )(src3, dst3, p1, zero_rows)


def _node1_body(spd_ref, p1_ref, q_ref, wab_ref, bn1_ref, be1_ref,
                x1_ref, g1_ref, g2_ref):
    sfull = spd_ref[0] + spd_ref[1]
    deg = sfull[:, HID:HID + 1]
    denom = jnp.maximum(deg, 1.0)
    p = p1_ref[:, :HID]
    x1 = jnp.maximum((deg * p - sfull[:, :HID]) / denom + bn1_ref[0, :], 0.0)
    g12 = _DOT(x1, wab_ref[...])
    qv = q_ref[...]
    x1_ref[...] = x1
    g1_ref[...] = g12[:, :HID] - qv
    g2_ref[...] = g12[:, HID:] + qv + be1_ref[0, :]


def _node1(spd, p1, q, wab, bn1, be1):
    return pl.pallas_call(
        _node1_body,
        grid=(NPAD // BN,),
        in_specs=[
            pl.BlockSpec((NC, BN, PW), lambda i: (0, i, 0)),
            pl.BlockSpec((BN, PW), lambda i: (i, 0)),
            pl.BlockSpec((BN, HID), lambda i: (i, 0)),
            pl.BlockSpec((HID, 2 * HID), lambda i: (0, 0)),
            pl.BlockSpec((1, HID), lambda i: (0, 0)),
            pl.BlockSpec((1, HID), lambda i: (0, 0)),
        ],
        out_specs=[
            pl.BlockSpec((BN, HID), lambda i: (i, 0)),
            pl.BlockSpec((BN, HID), lambda i: (i, 0)),
            pl.BlockSpec((BN, HID), lambda i: (i, 0)),
        ],
        out_shape=[
            jax.ShapeDtypeStruct((NPAD, HID), jnp.float32),
            jax.ShapeDtypeStruct((NPAD, HID), jnp.float32),
            jax.ShapeDtypeStruct((NPAD, HID), jnp.float32),
        ],
    )(spd, p1, q, wab, bn1, be1)


def _relu_rows(e_ref):
    def rows(i, carry):
        r = i * 4
        for t in range(4):
            for h in (0, 16):
                sl = pl.ds(h, 16)
                e_ref[r + t, sl] = jnp.maximum(e_ref[r + t, sl], 0.0)
        return carry

    lax.fori_loop(0, CHUNK // 4, rows, 0)


def _sc_edge1_body(src_hbm, dst_hbm, g1_hbm, g2_hbm, zero_hbm,
                   e1_hbm, out_hbm,
                   src_v, dst_v, e_v, zbuf_v, acc_s, sga, sgb, sst, ssc):
    c = lax.axis_index("c")
    s = lax.axis_index("s")
    r0 = s * ZROWS
    nmy, cbase = _my_chunks(c, s)

    pltpu.sync_copy(zero_hbm, zbuf_v)
    pltpu.sync_copy(zbuf_v, acc_s.at[pl.ds(r0, ZROWS)])
    pltpu.sync_copy(src_hbm.at[pl.ds(cbase, IDXROWS)], src_v)
    pltpu.sync_copy(dst_hbm.at[pl.ds(cbase, IDXROWS)], dst_v)
    plsc.subcore_barrier()

    def group(g, carry):
        j0 = g * RING
        for t in range(RING):
            pltpu.async_copy(g1_hbm.at[src_v.at[j0 + t]], e_v.at[t],
                             sga.at[t])
        for t in range(RING):
            pltpu.make_async_copy(g1_hbm.at[src_v.at[j0 + t]], e_v.at[t],
                                  sga.at[t]).wait()
            pltpu.async_copy(g2_hbm.at[dst_v.at[j0 + t]], e_v.at[t],
                             sgb.at[t], add=True)
        for t in range(RING):
            pltpu.make_async_copy(g2_hbm.at[dst_v.at[j0 + t]], e_v.at[t],
                                  sgb.at[t]).wait()
            _relu_rows(e_v.at[t])
            pltpu.async_copy(
                e_v.at[t],
                e1_hbm.at[pl.ds((cbase + j0 + t) * CHUNK, CHUNK)],
                sst.at[t])
            pltpu.async_copy(e_v.at[t], acc_s.at[dst_v.at[j0 + t]],
                             ssc.at[t], add=True)
        for t in range(RING):
            pltpu.make_async_copy(
                e_v.at[t],
                e1_hbm.at[pl.ds((cbase + j0 + t) * CHUNK, CHUNK)],
                sst.at[t]).wait()
            pltpu.make_async_copy(e_v.at[t], acc_s.at[dst_v.at[j0 + t]],
                                  ssc.at[t]).wait()
        return carry

    lax.fori_loop(0, nmy // RING, group, 0)
    plsc.subcore_barrier()
    pltpu.sync_copy(acc_s.at[pl.ds(r0, ZROWS)], zbuf_v)
    pltpu.sync_copy(zbuf_v, out_hbm.at[c, pl.ds(r0, ZROWS)])


def _sc_edge1(src3, dst3, g1, g2, zero_rows):
    return pl.kernel(
        _sc_edge1_body,
        out_type=[
            jax.ShapeDtypeStruct((EPAD, HID), jnp.float32),
            jax.ShapeDtypeStruct((NC, NPAD, HID), jnp.float32),
        ],
        mesh=_MESH,
        compiler_params=pltpu.CompilerParams(use_tc_tiling_on_sc=False),
        scratch_types=[
            pltpu.VMEM((IDXROWS, CHUNK), jnp.int32),
            pltpu.VMEM((IDXROWS, CHUNK), jnp.int32),
            pltpu.VMEM((RING, CHUNK, HID), jnp.float32),
            pltpu.VMEM((ZROWS, HID), jnp.float32),
            pltpu.VMEM_SHARED((NPAD, HID), jnp.float32),
            pltpu.SemaphoreType.DMA((RING,)),
            pltpu.SemaphoreType.DMA((RING,)),
            pltpu.SemaphoreType.DMA((RING,)),
            pltpu.SemaphoreType.DMA((RING,)),
        ],
    )(src3, dst3, g1, g2, zero_rows)


def _node2_body(sed_ref, spd_ref, x1_ref, wn2_ref, wab2_ref, bn2_ref, be2_ref,
                g3_ref, g4_ref):
    se = sed_ref[0] + sed_ref[1]
    deg = spd_ref[0][:, HID:HID + 1] + spd_ref[1][:, HID:HID + 1]
    denom = jnp.maximum(deg, 1.0)
    agg2 = se / denom
    cat = jnp.concatenate([x1_ref[...], agg2], axis=1)
    x2 = jnp.maximum(_DOT(cat, wn2_ref[...]) + bn2_ref[0, :], 0.0)
    g34 = _DOT(x2, wab2_ref[...])
    g3_ref[...] = g34[:, :HID]
    g4_ref[...] = g34[:, HID:] + be2_ref[0, :]


def _node2(sed, spd, x1, wn2, wab2, bn2, be2):
    return pl.pallas_call(
        _node2_body,
        grid=(NPAD // BN,),
        in_specs=[
            pl.BlockSpec((NC, BN, HID), lambda i: (0, i, 0)),
            pl.BlockSpec((NC, BN, PW), lambda i: (0, i, 0)),
            pl.BlockSpec((BN, HID), lambda i: (i, 0)),
            pl.BlockSpec((2 * HID, HID), lambda i: (0, 0)),
            pl.BlockSpec((HID, 2 * HID), lambda i: (0, 0)),
            pl.BlockSpec((1, HID), lambda i: (0, 0)),
            pl.BlockSpec((1, HID), lambda i: (0, 0)),
        ],
        out_specs=[
            pl.BlockSpec((BN, HID), lambda i: (i, 0)),
            pl.BlockSpec((BN, HID), lambda i: (i, 0)),
        ],
        out_shape=[
            jax.ShapeDtypeStruct((NPAD, HID), jnp.float32),
            jax.ShapeDtypeStruct((NPAD, HID), jnp.float32),
        ],
    )(sed, spd, x1, wn2, wab2, bn2, be2)


def _sc_edge2_body(src_hbm, dst_hbm, g3_hbm, g4_hbm, z2_hbm,
                   src_v, dst_v, e_v, sga, sgb, sst):
    c = lax.axis_index("c")
    s = lax.axis_index("s")
    nmy, cbase = _my_chunks(c, s)

    pltpu.sync_copy(src_hbm.at[pl.ds(cbase, IDXROWS)], src_v)
    pltpu.sync_copy(dst_hbm.at[pl.ds(cbase, IDXROWS)], dst_v)

    def group(g, carry):
        j0 = g * RING
        for t in range(RING):
            pltpu.async_copy(g3_hbm.at[src_v.at[j0 + t]], e_v.at[t],
                             sga.at[t])
        for t in range(RING):
            pltpu.make_async_copy(g3_hbm.at[src_v.at[j0 + t]], e_v.at[t],
                                  sga.at[t]).wait()
            pltpu.async_copy(g4_hbm.at[dst_v.at[j0 + t]], e_v.at[t],
                             sgb.at[t], add=True)
        for t in range(RING):
            pltpu.make_async_copy(g4_hbm.at[dst_v.at[j0 + t]], e_v.at[t],
                                  sgb.at[t]).wait()
            pltpu.async_copy(
                e_v.at[t],
                z2_hbm.at[pl.ds((cbase + j0 + t) * CHUNK, CHUNK)],
                sst.at[t])
        for t in range(RING):
            pltpu.make_async_copy(
                e_v.at[t],
                z2_hbm.at[pl.ds((cbase + j0 + t) * CHUNK, CHUNK)],
                sst.at[t]).wait()
        return carry

    lax.fori_loop(0, nmy // RING, group, 0)


def _sc_edge2(src3, dst3, g3, g4):
    return pl.kernel(
        _sc_edge2_body,
        out_type=jax.ShapeDtypeStruct((EPAD, HID), jnp.float32),
        mesh=_MESH,
        compiler_params=pltpu.CompilerParams(use_tc_tiling_on_sc=False),
        scratch_types=[
            pltpu.VMEM((IDXROWS, CHUNK), jnp.int32),
            pltpu.VMEM((IDXROWS, CHUNK), jnp.int32),
            pltpu.VMEM((RING, CHUNK, HID), jnp.float32),
            pltpu.SemaphoreType.DMA((RING,)),
            pltpu.SemaphoreType.DMA((RING,)),
            pltpu.SemaphoreType.DMA((RING,)),
        ],
    )(src3, dst3, g3, g4)


def _final_body(z2_ref, e1_ref, c2_ref, wc_ref, bc_ref, et_ref,
                pred_ref, loss_ref, s_ref):
    i = pl.program_id(0)
    t = _DOT(e1_ref[...], c2_ref[...])
    e2 = jnp.maximum(z2_ref[...] + t, 0.0)
    logit = _DOT(e2, wc_ref[...]) + bc_ref[0, 0]
    pred = 1.0 / (1.0 + jnp.exp(-logit))
    pred_ref[...] = pred

    @pl.when(i == 0)
    def _():
        for k in range(5):
            s_ref[k] = 0.0

    @pl.when(i < VALID_BLOCKS)
    def _():
        p2 = pred * pred
        pc = jnp.clip(pred, 1e-7, 1.0 - 1e-7)
        y = et_ref[...]
        bce = -(y * jnp.log(pc) + (1.0 - y) * jnp.log(1.0 - pc))
        s_ref[0] += jnp.sum(pred)
        s_ref[1] += jnp.sum(p2)
        s_ref[2] += jnp.sum(p2 * pred)
        s_ref[3] += jnp.sum(p2 * p2)
        s_ref[4] += jnp.sum(bce)

    @pl.when(i == pl.num_programs(0) - 1)
    def _():
        e = float(N_EDGES)
        m1 = s_ref[0] / e
        m2 = s_ref[1] / e
        m3 = s_ref[2] / e
        m4 = s_ref[3] / e
        msq = m1 * m1
        cm4 = m4 - 4.0 * m1 * m3 + 6.0 * msq * m2 - 3.0 * msq * msq
        cm4 = jnp.maximum(cm4, 0.0)
        reg = -jnp.sqrt(jnp.sqrt(cm4))
        loss_ref[0, 0] = s_ref[4] / e + 0.1 * reg


def _final(z2, e1, c2, wc, bc, et):
    return pl.pallas_call(
        _final_body,
        grid=(EPAD // BE,),
        in_specs=[
            pl.BlockSpec((BE, HID), lambda i: (i, 0)),
            pl.BlockSpec((BE, HID), lambda i: (i, 0)),
            pl.BlockSpec((HID, HID), lambda i: (0, 0)),
            pl.BlockSpec((HID, 1), lambda i: (0, 0)),
            pl.BlockSpec((1, 1), lambda i: (0, 0), memory_space=pltpu.SMEM),
            pl.BlockSpec((BE, 1), lambda i: (i, 0)),
        ],
        out_specs=[
            pl.BlockSpec((BE, 1), lambda i: (i, 0)),
            pl.BlockSpec((1, 1), lambda i: (0, 0), memory_space=pltpu.SMEM),
        ],
        out_shape=[
            jax.ShapeDtypeStruct((EPAD, 1), jnp.float32),
            jax.ShapeDtypeStruct((1, 1), jnp.float32),
        ],
        scratch_shapes=[pltpu.SMEM((8,), jnp.float32)],
    )(z2, e1, c2, wc, bc, et)


def kernel(x, edge_index, edge_type,
           W_n1, b_n1, W_e1, b_e1, W_n2, b_n2, W_e2, b_e2, W_c, b_c):
    f32 = jnp.float32
    src3 = jnp.pad(edge_index[0], (0, EPAD_IDX - N_EDGES),
                   constant_values=N_NODES).reshape(TOTC_PAD, CHUNK)
    dst3 = jnp.pad(edge_index[1], (0, EPAD_IDX - N_EDGES),
                   constant_values=N_NODES).reshape(TOTC_PAD, CHUNK)
    x_pad = jnp.pad(x, ((0, NPAD - N_NODES), (0, 0)))
    w_cat = jnp.concatenate([W_n1[DIM:], W_e1[2 * HID:]], axis=1)

    p1, q = _pq_tables(x_pad, w_cat)

    zero_p = jnp.zeros((ZROWS, PW), f32)
    spd = _sc_seg_p(src3, dst3, p1, zero_p)

    wab = jnp.concatenate([W_e1[:HID], W_e1[HID:2 * HID]], axis=1)
    x1, g1, g2 = _node1(spd, p1, q, wab,
                        b_n1.reshape(1, HID), b_e1.reshape(1, HID))

    zero_h = jnp.zeros((ZROWS, HID), f32)
    e1, sed = _sc_edge1(src3, dst3, g1, g2, zero_h)

    wab2 = jnp.concatenate([W_e2[:HID], W_e2[HID:2 * HID]], axis=1)
    g3, g4 = _node2(sed, spd, x1, W_n2, wab2,
                    b_n2.reshape(1, HID), b_e2.reshape(1, HID))

    z2 = _sc_edge2(src3, dst3, g3, g4)

    et = jnp.pad(edge_type.astype(f32), (0, EPAD - N_EDGES)).reshape(EPAD, 1)
    pred_pad, loss = _final(z2, e1, W_e2[2 * HID:], W_c,
                            b_c.reshape(1, 1), et)
    return pred_pad[:N_EDGES, 0], loss[0, 0]


# speedup vs baseline: 1.1685x; 1.0770x over previous
import functools

import jax
import jax.numpy as jnp
from jax import lax
from jax.experimental import pallas as pl
from jax.experimental.pallas import tpu as pltpu
from jax.experimental.pallas import tpu_sc as plsc

N_NODES = 10000
N_EDGES = 160000
DIM = 256
HID = 32

NC = 2
NS = 16
CHUNK = 128
TOTC = 1280
EPAD = TOTC * CHUNK
NPAD = 10240
PW = 48
ZROWS = NPAD // NS
RING = 4

FAST_CORE = 1
C_FAST = 52
C_SLOW = 28
IDXROWS = C_FAST
TOTC_PAD = TOTC + IDXROWS
EPAD_IDX = TOTC_PAD * CHUNK

BN = 1280
BE = 1280
VALID_BLOCKS = N_EDGES // BE

_MESH = plsc.VectorSubcoreMesh(
    core_axis_name="c", subcore_axis_name="s", num_cores=NC, num_subcores=NS)

_DOT = functools.partial(
    jnp.dot, preferred_element_type=jnp.float32, precision=lax.Precision.HIGHEST)


def _my_chunks(c, s):
    is_fast = c == FAST_CORE
    nmy = jnp.where(is_fast, C_FAST, C_SLOW)
    cbase = jnp.where(is_fast, s * C_FAST, NS * C_FAST + s * C_SLOW)
    return nmy, cbase


def _pq_body(x_ref, w_ref, p1_ref, q_ref):
    pq = _DOT(x_ref[...], w_ref[...])
    ones = jnp.ones((BN, 1), jnp.float32)
    zeros = jnp.zeros((BN, PW - HID - 1), jnp.float32)
    p1_ref[...] = jnp.concatenate([pq[:, :HID], ones, zeros], axis=1)
    q_ref[...] = pq[:, HID:]


def _pq_tables(x_pad, w_cat):
    return pl.pallas_call(
        _pq_body,
        grid=(NPAD // BN,),
        in_specs=[
            pl.BlockSpec((BN, DIM), lambda i: (i, 0)),
            pl.BlockSpec((DIM, 2 * HID), lambda i: (0, 0)),
        ],
        out_specs=[
            pl.BlockSpec((BN, PW), lambda i: (i, 0)),
            pl.BlockSpec((BN, HID), lambda i: (i, 0)),
        ],
        out_shape=[
            jax.ShapeDtypeStruct((NPAD, PW), jnp.float32),
            jax.ShapeDtypeStruct((NPAD, HID), jnp.float32),
        ],
    )(x_pad, w_cat)


def _sc_seg_p_body(src_hbm, dst_hbm, p1_hbm, zero_hbm, out_hbm,
                   src_v, dst_v, rows_v, zbuf_v, acc_s, sg, ss):
    c = lax.axis_index("c")
    s = lax.axis_index("s")
    r0 = s * ZROWS
    nmy, cbase = _my_chunks(c, s)

    pltpu.sync_copy(zero_hbm, zbuf_v)
    pltpu.sync_copy(zbuf_v, acc_s.at[pl.ds(r0, ZROWS)])
    pltpu.sync_copy(src_hbm.at[pl.ds(cbase, IDXROWS)], src_v)
    pltpu.sync_copy(dst_hbm.at[pl.ds(cbase, IDXROWS)], dst_v)
    plsc.subcore_barrier()

    def group(g, carry):
        j0 = g * RING
        for t in range(RING):
            pltpu.async_copy(p1_hbm.at[src_v.at[j0 + t]], rows_v.at[t],
                             sg.at[t])
        for t in range(RING):
            pltpu.make_async_copy(p1_hbm.at[src_v.at[j0 + t]], rows_v.at[t],
                                  sg.at[t]).wait()
            pltpu.async_copy(rows_v.at[t], acc_s.at[dst_v.at[j0 + t]],
                             ss.at[t], add=True)
        for t in range(RING):
            pltpu.make_async_copy(rows_v.at[t], acc_s.at[dst_v.at[j0 + t]],
                                  ss.at[t]).wait()
        return carry

    lax.fori_loop(0, nmy // RING, group, 0)
    plsc.subcore_barrier()
    pltpu.sync_copy(acc_s.at[pl.ds(r0, ZROWS)], zbuf_v)
    pltpu.sync_copy(zbuf_v, out_hbm.at[c, pl.ds(r0, ZROWS)])


def _sc_seg_p(src3, dst3, p1, zero_rows):
    return pl.kernel(
        _sc_seg_p_body,
        out_type=jax.ShapeDtypeStruct((NC, NPAD, PW), jnp.float32),
        mesh=_MESH,
        compiler_params=pltpu.CompilerParams(use_tc_tiling_on_sc=False),
        scratch_types=[
            pltpu.VMEM((IDXROWS, CHUNK), jnp.int32),
            pltpu.VMEM((IDXROWS, CHUNK), jnp.int32),
            pltpu.VMEM((RING, CHUNK, PW), jnp.float32),
            pltpu.VMEM((ZROWS, PW), jnp.float32),
            pltpu.VMEM_SHARED((NPAD, PW), jnp.float32),
            pltpu.SemaphoreType.DMA((RING,)),
            pltpu.SemaphoreType.DMA((RING,)),
        ],
    )(src3, dst3, p1, zero_rows)


def _node1_body(spd_ref, p1_ref, q_ref, wab_ref, bn1_ref, be1_ref,
                x1_ref, g1_ref, g2_ref):
    sfull = spd_ref[0] + spd_ref[1]
    deg = sfull[:, HID:HID + 1]
    denom = jnp.maximum(deg, 1.0)
    p = p1_ref[:, :HID]
    x1 = jnp.maximum((deg * p - sfull[:, :HID]) / denom + bn1_ref[0, :], 0.0)
    g12 = _DOT(x1, wab_ref[...])
    qv = q_ref[...]
    x1_ref[...] = x1
    g1_ref[...] = g12[:, :HID] - qv
    g2_ref[...] = g12[:, HID:] + qv + be1_ref[0, :]


def _node1(spd, p1, q, wab, bn1, be1):
    return pl.pallas_call(
        _node1_body,
        grid=(NPAD // BN,),
        in_specs=[
            pl.BlockSpec((NC, BN, PW), lambda i: (0, i, 0)),
            pl.BlockSpec((BN, PW), lambda i: (i, 0)),
            pl.BlockSpec((BN, HID), lambda i: (i, 0)),
            pl.BlockSpec((HID, 2 * HID), lambda i: (0, 0)),
            pl.BlockSpec((1, HID), lambda i: (0, 0)),
            pl.BlockSpec((1, HID), lambda i: (0, 0)),
        ],
        out_specs=[
            pl.BlockSpec((BN, HID), lambda i: (i, 0)),
            pl.BlockSpec((BN, HID), lambda i: (i, 0)),
            pl.BlockSpec((BN, HID), lambda i: (i, 0)),
        ],
        out_shape=[
            jax.ShapeDtypeStruct((NPAD, HID), jnp.float32),
            jax.ShapeDtypeStruct((NPAD, HID), jnp.float32),
            jax.ShapeDtypeStruct((NPAD, HID), jnp.float32),
        ],
    )(spd, p1, q, wab, bn1, be1)


def _relu_rows(e_ref):
    def rows(i, carry):
        r = i * 4
        for t in range(4):
            for h in (0, 16):
                sl = pl.ds(h, 16)
                e_ref[r + t, sl] = jnp.maximum(e_ref[r + t, sl], 0.0)
        return carry

    lax.fori_loop(0, CHUNK // 4, rows, 0)


def _sc_edge1_body(src_hbm, dst_hbm, g1_hbm, g2_hbm, zero_hbm,
                   e1_hbm, out_hbm,
                   src_v, dst_v, e_v, zbuf_v, acc_s, sga, sgb, sst, ssc):
    c = lax.axis_index("c")
    s = lax.axis_index("s")
    r0 = s * ZROWS
    nmy, cbase = _my_chunks(c, s)

    pltpu.sync_copy(zero_hbm, zbuf_v)
    pltpu.sync_copy(zbuf_v, acc_s.at[pl.ds(r0, ZROWS)])
    pltpu.sync_copy(src_hbm.at[pl.ds(cbase, IDXROWS)], src_v)
    pltpu.sync_copy(dst_hbm.at[pl.ds(cbase, IDXROWS)], dst_v)
    plsc.subcore_barrier()

    def group(g, carry):
        j0 = g * RING
        for t in range(RING):
            pltpu.async_copy(g1_hbm.at[src_v.at[j0 + t]], e_v.at[t],
                             sga.at[t])
        for t in range(RING):
            pltpu.make_async_copy(g1_hbm.at[src_v.at[j0 + t]], e_v.at[t],
                                  sga.at[t]).wait()
            pltpu.async_copy(g2_hbm.at[dst_v.at[j0 + t]], e_v.at[t],
                             sgb.at[t], add=True)
        for t in range(RING):
            pltpu.make_async_copy(g2_hbm.at[dst_v.at[j0 + t]], e_v.at[t],
                                  sgb.at[t]).wait()
            _relu_rows(e_v.at[t])
            pltpu.async_copy(
                e_v.at[t],
                e1_hbm.at[pl.ds((cbase + j0 + t) * CHUNK, CHUNK)],
                sst.at[t])
            pltpu.async_copy(e_v.at[t], acc_s.at[dst_v.at[j0 + t]],
                             ssc.at[t], add=True)
        for t in range(RING):
            pltpu.make_async_copy(
                e_v.at[t],
                e1_hbm.at[pl.ds((cbase + j0 + t) * CHUNK, CHUNK)],
                sst.at[t]).wait()
            pltpu.make_async_copy(e_v.at[t], acc_s.at[dst_v.at[j0 + t]],
                                  ssc.at[t]).wait()
        return carry

    lax.fori_loop(0, nmy // RING, group, 0)
    plsc.subcore_barrier()
    pltpu.sync_copy(acc_s.at[pl.ds(r0, ZROWS)], zbuf_v)
    pltpu.sync_copy(zbuf_v, out_hbm.at[c, pl.ds(r0, ZROWS)])


def _sc_edge1(src3, dst3, g1, g2, zero_rows):
    return pl.kernel(
        _sc_edge1_body,
        out_type=[
            jax.ShapeDtypeStruct((EPAD, HID), jnp.float32),
            jax.ShapeDtypeStruct((NC, NPAD, HID), jnp.float32),
        ],
        mesh=_MESH,
        compiler_params=pltpu.CompilerParams(use_tc_tiling_on_sc=False),
        scratch_types=[
            pltpu.VMEM((IDXROWS, CHUNK), jnp.int32),
            pltpu.VMEM((IDXROWS, CHUNK), jnp.int32),
            pltpu.VMEM((RING, CHUNK, HID), jnp.float32),
            pltpu.VMEM((ZROWS, HID), jnp.float32),
            pltpu.VMEM_SHARED((NPAD, HID), jnp.float32),
            pltpu.SemaphoreType.DMA((RING,)),
            pltpu.SemaphoreType.DMA((RING,)),
            pltpu.SemaphoreType.DMA((RING,)),
            pltpu.SemaphoreType.DMA((RING,)),
        ],
    )(src3, dst3, g1, g2, zero_rows)


def _node2_body(sed_ref, spd_ref, x1_ref, wn2_ref, wab2_ref, bn2_ref, be2_ref,
                g3_ref, g4_ref):
    se = sed_ref[0] + sed_ref[1]
    deg = spd_ref[0][:, HID:HID + 1] + spd_ref[1][:, HID:HID + 1]
    denom = jnp.maximum(deg, 1.0)
    agg2 = se / denom
    cat = jnp.concatenate([x1_ref[...], agg2], axis=1)
    x2 = jnp.maximum(_DOT(cat, wn2_ref[...]) + bn2_ref[0, :], 0.0)
    g34 = _DOT(x2, wab2_ref[...])
    g3_ref[...] = g34[:, :HID]
    g4_ref[...] = g34[:, HID:] + be2_ref[0, :]


def _node2(sed, spd, x1, wn2, wab2, bn2, be2):
    return pl.pallas_call(
        _node2_body,
        grid=(NPAD // BN,),
        in_specs=[
            pl.BlockSpec((NC, BN, HID), lambda i: (0, i, 0)),
            pl.BlockSpec((NC, BN, PW), lambda i: (0, i, 0)),
            pl.BlockSpec((BN, HID), lambda i: (i, 0)),
            pl.BlockSpec((2 * HID, HID), lambda i: (0, 0)),
            pl.BlockSpec((HID, 2 * HID), lambda i: (0, 0)),
            pl.BlockSpec((1, HID), lambda i: (0, 0)),
            pl.BlockSpec((1, HID), lambda i: (0, 0)),
        ],
        out_specs=[
            pl.BlockSpec((BN, HID), lambda i: (i, 0)),
            pl.BlockSpec((BN, HID), lambda i: (i, 0)),
        ],
        out_shape=[
            jax.ShapeDtypeStruct((NPAD, HID), jnp.float32),
            jax.ShapeDtypeStruct((NPAD, HID), jnp.float32),
        ],
    )(sed, spd, x1, wn2, wab2, bn2, be2)


def _sc_edge2_body(src_hbm, dst_hbm, g3_hbm, g4_hbm, z2_hbm,
                   src_v, dst_v, e_v, sga, sgb, sst):
    c = lax.axis_index("c")
    s = lax.axis_index("s")
    nmy, cbase = _my_chunks(c, s)

    pltpu.sync_copy(src_hbm.at[pl.ds(cbase, IDXROWS)], src_v)
    pltpu.sync_copy(dst_hbm.at[pl.ds(cbase, IDXROWS)], dst_v)

    def group(g, carry):
        j0 = g * RING
        for t in range(RING):
            pltpu.async_copy(g3_hbm.at[src_v.at[j0 + t]], e_v.at[t],
                             sga.at[t])
        for t in range(RING):
            pltpu.make_async_copy(g3_hbm.at[src_v.at[j0 + t]], e_v.at[t],
                                  sga.at[t]).wait()
            pltpu.async_copy(g4_hbm.at[dst_v.at[j0 + t]], e_v.at[t],
                             sgb.at[t], add=True)
        for t in range(RING):
            pltpu.make_async_copy(g4_hbm.at[dst_v.at[j0 + t]], e_v.at[t],
                                  sgb.at[t]).wait()
            pltpu.async_copy(
                e_v.at[t],
                z2_hbm.at[pl.ds((cbase + j0 + t) * CHUNK, CHUNK)],
                sst.at[t])
        for t in range(RING):
            pltpu.make_async_copy(
                e_v.at[t],
                z2_hbm.at[pl.ds((cbase + j0 + t) * CHUNK, CHUNK)],
                sst.at[t]).wait()
        return carry

    lax.fori_loop(0, nmy // RING, group, 0)


def _sc_edge2(src3, dst3, g3, g4):
    return pl.kernel(
        _sc_edge2_body,
        out_type=jax.ShapeDtypeStruct((EPAD, HID), jnp.float32),
        mesh=_MESH,
        compiler_params=pltpu.CompilerParams(use_tc_tiling_on_sc=False),
        scratch_types=[
            pltpu.VMEM((IDXROWS, CHUNK), jnp.int32),
            pltpu.VMEM((IDXROWS, CHUNK), jnp.int32),
            pltpu.VMEM((RING, CHUNK, HID), jnp.float32),
            pltpu.SemaphoreType.DMA((RING,)),
            pltpu.SemaphoreType.DMA((RING,)),
            pltpu.SemaphoreType.DMA((RING,)),
        ],
    )(src3, dst3, g3, g4)


def _logits_body(z2_ref, e1_ref, c2_ref, wc_ref, bc_ref, lg_ref):
    t = _DOT(e1_ref[...], c2_ref[...])
    e2 = jnp.maximum(z2_ref[...] + t, 0.0)
    lg_ref[...] = _DOT(e2, wc_ref[...]) + bc_ref[0, 0]


def _logits(z2, e1, c2, wc, bc):
    return pl.pallas_call(
        _logits_body,
        grid=(EPAD // BE,),
        in_specs=[
            pl.BlockSpec((BE, HID), lambda i: (i, 0)),
            pl.BlockSpec((BE, HID), lambda i: (i, 0)),
            pl.BlockSpec((HID, HID), lambda i: (0, 0)),
            pl.BlockSpec((HID, 1), lambda i: (0, 0)),
            pl.BlockSpec((1, 1), lambda i: (0, 0), memory_space=pltpu.SMEM),
        ],
        out_specs=pl.BlockSpec((BE, 1), lambda i: (i, 0)),
        out_shape=jax.ShapeDtypeStruct((EPAD, 1), jnp.float32),
    )(z2, e1, c2, wc, bc)


LROWS = EPAD // BE
LBLK = 8
VALID_ROWS = N_EDGES // BE


def _loss_body(lg_ref, et_ref, pred_ref, loss_ref, acc_ref):
    i = pl.program_id(0)
    lg = lg_ref[...]
    pred = 1.0 / (1.0 + jnp.exp(-lg))
    pred_ref[...] = pred

    @pl.when(i == 0)
    def _():
        acc_ref[...] = jnp.zeros_like(acc_ref)

    row = jax.lax.broadcasted_iota(jnp.int32, (LBLK, BE), 0) + i * LBLK
    mask = jnp.where(row < VALID_ROWS, 1.0, 0.0)
    pm = pred * mask
    p2 = pm * pred
    pc = jnp.clip(pred, 1e-7, 1.0 - 1e-7)
    y = et_ref[...]
    bce = -(y * jnp.log(pc) + (1.0 - y) * jnp.log(1.0 - pc)) * mask

    def fold(v):
        r = v.reshape(LBLK, BE // 128, 128)
        return jnp.sum(r, axis=1)

    acc_ref[0] += fold(pm)
    acc_ref[1] += fold(p2)
    acc_ref[2] += fold(p2 * pred)
    acc_ref[3] += fold(p2 * p2)
    acc_ref[4] += fold(bce)

    @pl.when(i == pl.num_programs(0) - 1)
    def _():
        e = float(N_EDGES)
        m1 = jnp.sum(acc_ref[0]) / e
        m2 = jnp.sum(acc_ref[1]) / e
        m3 = jnp.sum(acc_ref[2]) / e
        m4 = jnp.sum(acc_ref[3]) / e
        msq = m1 * m1
        cm4 = m4 - 4.0 * m1 * m3 + 6.0 * msq * m2 - 3.0 * msq * msq
        cm4 = jnp.maximum(cm4, 0.0)
        reg = -jnp.sqrt(jnp.sqrt(cm4))
        loss_ref[0, 0] = jnp.sum(acc_ref[4]) / e + 0.1 * reg


def _loss(lg2, et2):
    return pl.pallas_call(
        _loss_body,
        grid=(LROWS // LBLK,),
        in_specs=[
            pl.BlockSpec((LBLK, BE), lambda i: (i, 0)),
            pl.BlockSpec((LBLK, BE), lambda i: (i, 0)),
        ],
        out_specs=[
            pl.BlockSpec((LBLK, BE), lambda i: (i, 0)),
            pl.BlockSpec((1, 1), lambda i: (0, 0), memory_space=pltpu.SMEM),
        ],
        out_shape=[
            jax.ShapeDtypeStruct((LROWS, BE), jnp.float32),
            jax.ShapeDtypeStruct((1, 1), jnp.float32),
        ],
        scratch_shapes=[pltpu.VMEM((5, LBLK, 128), jnp.float32)],
    )(lg2, et2)


def kernel(x, edge_index, edge_type,
           W_n1, b_n1, W_e1, b_e1, W_n2, b_n2, W_e2, b_e2, W_c, b_c):
    f32 = jnp.float32
    src3 = jnp.pad(edge_index[0], (0, EPAD_IDX - N_EDGES),
                   constant_values=N_NODES).reshape(TOTC_PAD, CHUNK)
    dst3 = jnp.pad(edge_index[1], (0, EPAD_IDX - N_EDGES),
                   constant_values=N_NODES).reshape(TOTC_PAD, CHUNK)
    x_pad = jnp.pad(x, ((0, NPAD - N_NODES), (0, 0)))
    w_cat = jnp.concatenate([W_n1[DIM:], W_e1[2 * HID:]], axis=1)

    p1, q = _pq_tables(x_pad, w_cat)

    zero_p = jnp.zeros((ZROWS, PW), f32)
    spd = _sc_seg_p(src3, dst3, p1, zero_p)

    wab = jnp.concatenate([W_e1[:HID], W_e1[HID:2 * HID]], axis=1)
    x1, g1, g2 = _node1(spd, p1, q, wab,
                        b_n1.reshape(1, HID), b_e1.reshape(1, HID))

    zero_h = jnp.zeros((ZROWS, HID), f32)
    e1, sed = _sc_edge1(src3, dst3, g1, g2, zero_h)

    wab2 = jnp.concatenate([W_e2[:HID], W_e2[HID:2 * HID]], axis=1)
    g3, g4 = _node2(sed, spd, x1, W_n2, wab2,
                    b_n2.reshape(1, HID), b_e2.reshape(1, HID))

    z2 = _sc_edge2(src3, dst3, g3, g4)

    lg = _logits(z2, e1, W_e2[2 * HID:], W_c, b_c.reshape(1, 1))
    lg2 = lg.reshape(LROWS, BE)
    et2 = jnp.pad(edge_type.astype(f32), (0, EPAD - N_EDGES)).reshape(LROWS, BE)
    pred2, loss = _loss(lg2, et2)
    return pred2.reshape(EPAD)[:N_EDGES], loss[0, 0]


# speedup vs baseline: 1.3513x; 1.1565x over previous
import functools

import jax
import jax.numpy as jnp
from jax import lax
from jax.experimental import pallas as pl
from jax.experimental.pallas import tpu as pltpu
from jax.experimental.pallas import tpu_sc as plsc

N_NODES = 10000
N_EDGES = 160000
DIM = 256
HID = 32

NC = 2
NS = 16
CHUNK = 128
TOTC = 1280
EPAD = TOTC * CHUNK
NPAD = 10240
PW = 48
ZROWS = NPAD // NS
RING = 4

FAST_CORE = 1
C_FAST = 64
C_SLOW = 16
IDXROWS = C_FAST
TOTC_PAD = TOTC + IDXROWS
EPAD_IDX = TOTC_PAD * CHUNK

BN = 1280
BE = 1280
VALID_BLOCKS = N_EDGES // BE
LROWS = EPAD // BE
LBLK = 8
VALID_ROWS = N_EDGES // BE

_MESH = plsc.VectorSubcoreMesh(
    core_axis_name="c", subcore_axis_name="s", num_cores=NC, num_subcores=NS)

_DOT = functools.partial(
    jnp.dot, preferred_element_type=jnp.float32, precision=lax.Precision.HIGHEST)


def _my_chunks(c, s):
    is_fast = c == FAST_CORE
    nmy = jnp.where(is_fast, C_FAST, C_SLOW)
    cbase = jnp.where(is_fast, s * C_FAST, NS * C_FAST + s * C_SLOW)
    return nmy, cbase


def _pq_body(x_ref, w_ref, p1_ref, q_ref):
    pq = _DOT(x_ref[...], w_ref[...])
    ones = jnp.ones((BN, 1), jnp.float32)
    zeros = jnp.zeros((BN, PW - HID - 1), jnp.float32)
    p1_ref[...] = jnp.concatenate([pq[:, :HID], ones, zeros], axis=1)
    q_ref[...] = pq[:, HID:]


def _pq_tables(x_pad, w_cat):
    return pl.pallas_call(
        _pq_body,
        grid=(NPAD // BN,),
        in_specs=[
            pl.BlockSpec((BN, DIM), lambda i: (i, 0)),
            pl.BlockSpec((DIM, 2 * HID), lambda i: (0, 0)),
        ],
        out_specs=[
            pl.BlockSpec((BN, PW), lambda i: (i, 0)),
            pl.BlockSpec((BN, HID), lambda i: (i, 0)),
        ],
        out_shape=[
            jax.ShapeDtypeStruct((NPAD, PW), jnp.float32),
            jax.ShapeDtypeStruct((NPAD, HID), jnp.float32),
        ],
    )(x_pad, w_cat)


def _sc_seg_p_body(src_hbm, dst_hbm, p1_hbm, zero_hbm, out_hbm,
                   src_v, dst_v, rows_v, zbuf_v, acc_s, sg, ss):
    c = lax.axis_index("c")
    s = lax.axis_index("s")
    r0 = s * ZROWS
    nmy, cbase = _my_chunks(c, s)

    pltpu.sync_copy(zero_hbm, zbuf_v)
    pltpu.sync_copy(zbuf_v, acc_s.at[pl.ds(r0, ZROWS)])
    pltpu.sync_copy(src_hbm.at[pl.ds(cbase, IDXROWS)], src_v)
    pltpu.sync_copy(dst_hbm.at[pl.ds(cbase, IDXROWS)], dst_v)
    plsc.subcore_barrier()

    def group(g, carry):
        j0 = g * RING
        for t in range(RING):
            pltpu.async_copy(p1_hbm.at[src_v.at[j0 + t]], rows_v.at[t],
                             sg.at[t])
        for t in range(RING):
            pltpu.make_async_copy(p1_hbm.at[src_v.at[j0 + t]], rows_v.at[t],
                                  sg.at[t]).wait()
            pltpu.async_copy(rows_v.at[t], acc_s.at[dst_v.at[j0 + t]],
                             ss.at[t], add=True)
        for t in range(RING):
            pltpu.make_async_copy(rows_v.at[t], acc_s.at[dst_v.at[j0 + t]],
                                  ss.at[t]).wait()
        return carry

    lax.fori_loop(0, nmy // RING, group, 0)
    plsc.subcore_barrier()
    pltpu.sync_copy(acc_s.at[pl.ds(r0, ZROWS)], zbuf_v)
    pltpu.sync_copy(zbuf_v, out_hbm.at[c, pl.ds(r0, ZROWS)])


def _sc_seg_p(src3, dst3, p1, zero_rows):
    return pl.kernel(
        _sc_seg_p_body,
        out_type=jax.ShapeDtypeStruct((NC, NPAD, PW), jnp.float32),
        mesh=_MESH,
        compiler_params=pltpu.CompilerParams(use_tc_tiling_on_sc=False),
        scratch_types=[
            pltpu.VMEM((IDXROWS, CHUNK), jnp.int32),
            pltpu.VMEM((IDXROWS, CHUNK), jnp.int32),
            pltpu.VMEM((RING, CHUNK, PW), jnp.float32),
            pltpu.VMEM((ZROWS, PW), jnp.float32),
            pltpu.VMEM_SHARED((NPAD, PW), jnp.float32),
            pltpu.SemaphoreType.DMA((RING,)),
            pltpu.SemaphoreType.DMA((RING,)),
        ],
    )(src3, dst3, p1, zero_rows)


def _node1_body(spd_ref, p1_ref, q_ref, wab_ref, bn1_ref, be1_ref,
                x1_ref, g1_ref, g2_ref):
    sfull = spd_ref[0] + spd_ref[1]
    deg = sfull[:, HID:HID + 1]
    denom = jnp.maximum(deg, 1.0)
    p = p1_ref[:, :HID]
    x1 = jnp.maximum((deg * p - sfull[:, :HID]) / denom + bn1_ref[0, :], 0.0)
    g12 = _DOT(x1, wab_ref[...])
    qv = q_ref[...]
    x1_ref[...] = x1
    g1_ref[...] = g12[:, :HID] - qv
    g2_ref[...] = g12[:, HID:] + qv + be1_ref[0, :]


def _node1(spd, p1, q, wab, bn1, be1):
    return pl.pallas_call(
        _node1_body,
        grid=(NPAD // BN,),
        in_specs=[
            pl.BlockSpec((NC, BN, PW), lambda i: (0, i, 0)),
            pl.BlockSpec((BN, PW), lambda i: (i, 0)),
            pl.BlockSpec((BN, HID), lambda i: (i, 0)),
            pl.BlockSpec((HID, 2 * HID), lambda i: (0, 0)),
            pl.BlockSpec((1, HID), lambda i: (0, 0)),
            pl.BlockSpec((1, HID), lambda i: (0, 0)),
        ],
        out_specs=[
            pl.BlockSpec((BN, HID), lambda i: (i, 0)),
            pl.BlockSpec((BN, HID), lambda i: (i, 0)),
            pl.BlockSpec((BN, HID), lambda i: (i, 0)),
        ],
        out_shape=[
            jax.ShapeDtypeStruct((NPAD, HID), jnp.float32),
            jax.ShapeDtypeStruct((NPAD, HID), jnp.float32),
            jax.ShapeDtypeStruct((NPAD, HID), jnp.float32),
        ],
    )(spd, p1, q, wab, bn1, be1)


def _relu_rows(e_ref):
    def rows(i, carry):
        r = i * 4
        for t in range(4):
            for h in (0, 16):
                sl = pl.ds(h, 16)
                e_ref[r + t, sl] = jnp.maximum(e_ref[r + t, sl], 0.0)
        return carry

    lax.fori_loop(0, CHUNK // 4, rows, 0)


def _sc_edge1_body(src_hbm, dst_hbm, g1_hbm, g2_hbm, zero_hbm,
                   e1_hbm, out_hbm,
                   src_v, dst_v, e_v, zbuf_v, acc_s, sga, sgb, sst, ssc):
    c = lax.axis_index("c")
    s = lax.axis_index("s")
    r0 = s * ZROWS
    nmy, cbase = _my_chunks(c, s)

    pltpu.sync_copy(zero_hbm, zbuf_v)
    pltpu.sync_copy(zbuf_v, acc_s.at[pl.ds(r0, ZROWS)])
    pltpu.sync_copy(src_hbm.at[pl.ds(cbase, IDXROWS)], src_v)
    pltpu.sync_copy(dst_hbm.at[pl.ds(cbase, IDXROWS)], dst_v)
    plsc.subcore_barrier()

    def group(g, carry):
        j0 = g * RING
        for t in range(RING):
            pltpu.async_copy(g1_hbm.at[src_v.at[j0 + t]], e_v.at[t],
                             sga.at[t])
        for t in range(RING):
            pltpu.make_async_copy(g1_hbm.at[src_v.at[j0 + t]], e_v.at[t],
                                  sga.at[t]).wait()
            pltpu.async_copy(g2_hbm.at[dst_v.at[j0 + t]], e_v.at[t],
                             sgb.at[t], add=True)
        for t in range(RING):
            pltpu.make_async_copy(g2_hbm.at[dst_v.at[j0 + t]], e_v.at[t],
                                  sgb.at[t]).wait()
            _relu_rows(e_v.at[t])
            pltpu.async_copy(
                e_v.at[t],
                e1_hbm.at[pl.ds((cbase + j0 + t) * CHUNK, CHUNK)],
                sst.at[t])
            pltpu.async_copy(e_v.at[t], acc_s.at[dst_v.at[j0 + t]],
                             ssc.at[t], add=True)
        for t in range(RING):
            pltpu.make_async_copy(
                e_v.at[t],
                e1_hbm.at[pl.ds((cbase + j0 + t) * CHUNK, CHUNK)],
                sst.at[t]).wait()
            pltpu.make_async_copy(e_v.at[t], acc_s.at[dst_v.at[j0 + t]],
                                  ssc.at[t]).wait()
        return carry

    lax.fori_loop(0, nmy // RING, group, 0)
    plsc.subcore_barrier()
    pltpu.sync_copy(acc_s.at[pl.ds(r0, ZROWS)], zbuf_v)
    pltpu.sync_copy(zbuf_v, out_hbm.at[c, pl.ds(r0, ZROWS)])


def _sc_edge1(src3, dst3, g1, g2, zero_rows):
    return pl.kernel(
        _sc_edge1_body,
        out_type=[
            jax.ShapeDtypeStruct((EPAD, HID), jnp.float32),
            jax.ShapeDtypeStruct((NC, NPAD, HID), jnp.float32),
        ],
        mesh=_MESH,
        compiler_params=pltpu.CompilerParams(use_tc_tiling_on_sc=False),
        scratch_types=[
            pltpu.VMEM((IDXROWS, CHUNK), jnp.int32),
            pltpu.VMEM((IDXROWS, CHUNK), jnp.int32),
            pltpu.VMEM((RING, CHUNK, HID), jnp.float32),
            pltpu.VMEM((ZROWS, HID), jnp.float32),
            pltpu.VMEM_SHARED((NPAD, HID), jnp.float32),
            pltpu.SemaphoreType.DMA((RING,)),
            pltpu.SemaphoreType.DMA((RING,)),
            pltpu.SemaphoreType.DMA((RING,)),
            pltpu.SemaphoreType.DMA((RING,)),
        ],
    )(src3, dst3, g1, g2, zero_rows)


def _node2_body(sed_ref, spd_ref, x1_ref, wn2_ref, wab2_ref, bn2_ref, be2_ref,
                g3_ref, g4_ref):
    se = sed_ref[0] + sed_ref[1]
    deg = spd_ref[0][:, HID:HID + 1] + spd_ref[1][:, HID:HID + 1]
    denom = jnp.maximum(deg, 1.0)
    agg2 = se / denom
    cat = jnp.concatenate([x1_ref[...], agg2], axis=1)
    x2 = jnp.maximum(_DOT(cat, wn2_ref[...]) + bn2_ref[0, :], 0.0)
    g34 = _DOT(x2, wab2_ref[...])
    g3_ref[...] = g34[:, :HID]
    g4_ref[...] = g34[:, HID:] + be2_ref[0, :]


def _node2(sed, spd, x1, wn2, wab2, bn2, be2):
    return pl.pallas_call(
        _node2_body,
        grid=(NPAD // BN,),
        in_specs=[
            pl.BlockSpec((NC, BN, HID), lambda i: (0, i, 0)),
            pl.BlockSpec((NC, BN, PW), lambda i: (0, i, 0)),
            pl.BlockSpec((BN, HID), lambda i: (i, 0)),
            pl.BlockSpec((2 * HID, HID), lambda i: (0, 0)),
            pl.BlockSpec((HID, 2 * HID), lambda i: (0, 0)),
            pl.BlockSpec((1, HID), lambda i: (0, 0)),
            pl.BlockSpec((1, HID), lambda i: (0, 0)),
        ],
        out_specs=[
            pl.BlockSpec((BN, HID), lambda i: (i, 0)),
            pl.BlockSpec((BN, HID), lambda i: (i, 0)),
        ],
        out_shape=[
            jax.ShapeDtypeStruct((NPAD, HID), jnp.float32),
            jax.ShapeDtypeStruct((NPAD, HID), jnp.float32),
        ],
    )(sed, spd, x1, wn2, wab2, bn2, be2)


def _sc_edge2_body(src_hbm, dst_hbm, g3_hbm, g4_hbm, z2_hbm,
                   src_v, dst_v, e_v, sga, sgb, sst):
    c = lax.axis_index("c")
    s = lax.axis_index("s")
    nmy, cbase = _my_chunks(c, s)

    pltpu.sync_copy(src_hbm.at[pl.ds(cbase, IDXROWS)], src_v)
    pltpu.sync_copy(dst_hbm.at[pl.ds(cbase, IDXROWS)], dst_v)

    def group(g, carry):
        j0 = g * RING
        for t in range(RING):
            pltpu.async_copy(g3_hbm.at[src_v.at[j0 + t]], e_v.at[t],
                             sga.at[t])
        for t in range(RING):
            pltpu.make_async_copy(g3_hbm.at[src_v.at[j0 + t]], e_v.at[t],
                                  sga.at[t]).wait()
            pltpu.async_copy(g4_hbm.at[dst_v.at[j0 + t]], e_v.at[t],
                             sgb.at[t], add=True)
        for t in range(RING):
            pltpu.make_async_copy(g4_hbm.at[dst_v.at[j0 + t]], e_v.at[t],
                                  sgb.at[t]).wait()
            pltpu.async_copy(
                e_v.at[t],
                z2_hbm.at[pl.ds((cbase + j0 + t) * CHUNK, CHUNK)],
                sst.at[t])
        for t in range(RING):
            pltpu.make_async_copy(
                e_v.at[t],
                z2_hbm.at[pl.ds((cbase + j0 + t) * CHUNK, CHUNK)],
                sst.at[t]).wait()
        return carry

    lax.fori_loop(0, nmy // RING, group, 0)


def _sc_edge2(src3, dst3, g3, g4):
    return pl.kernel(
        _sc_edge2_body,
        out_type=jax.ShapeDtypeStruct((EPAD, HID), jnp.float32),
        mesh=_MESH,
        compiler_params=pltpu.CompilerParams(use_tc_tiling_on_sc=False),
        scratch_types=[
            pltpu.VMEM((IDXROWS, CHUNK), jnp.int32),
            pltpu.VMEM((IDXROWS, CHUNK), jnp.int32),
            pltpu.VMEM((RING, CHUNK, HID), jnp.float32),
            pltpu.SemaphoreType.DMA((RING,)),
            pltpu.SemaphoreType.DMA((RING,)),
            pltpu.SemaphoreType.DMA((RING,)),
        ],
    )(src3, dst3, g3, g4)


FBE = LBLK * BE


def _final_body(z2_ref, e1_ref, c2_ref, wct_ref, bc_ref, et_ref,
                pred_ref, loss_ref, acc_ref):
    i = pl.program_id(0)
    t = _DOT(e1_ref[...], c2_ref[...])
    e2 = jnp.maximum(z2_ref[...] + t, 0.0)
    lgt = lax.dot_general(wct_ref[...], e2, (((1,), (1,)), ((), ())),
                          precision=lax.Precision.HIGHEST,
                          preferred_element_type=jnp.float32)
    lg = lgt.reshape(LBLK, BE) + bc_ref[0, 0]
    pred = 1.0 / (1.0 + jnp.exp(-lg))
    pred_ref[...] = pred

    @pl.when(i == 0)
    def _():
        acc_ref[...] = jnp.zeros_like(acc_ref)

    row = jax.lax.broadcasted_iota(jnp.int32, (LBLK, BE), 0) + i * LBLK
    mask = jnp.where(row < VALID_ROWS, 1.0, 0.0)
    pm = pred * mask
    p2 = pm * pred
    pc = jnp.clip(pred, 1e-7, 1.0 - 1e-7)
    y = et_ref[...]
    bce = -(y * jnp.log(pc) + (1.0 - y) * jnp.log(1.0 - pc)) * mask

    def fold(v):
        r = v.reshape(LBLK, BE // 128, 128)
        return jnp.sum(r, axis=1)

    acc_ref[0] += fold(pm)
    acc_ref[1] += fold(p2)
    acc_ref[2] += fold(p2 * pred)
    acc_ref[3] += fold(p2 * p2)
    acc_ref[4] += fold(bce)

    @pl.when(i == pl.num_programs(0) - 1)
    def _():
        e = float(N_EDGES)
        m1 = jnp.sum(acc_ref[0]) / e
        m2 = jnp.sum(acc_ref[1]) / e
        m3 = jnp.sum(acc_ref[2]) / e
        m4 = jnp.sum(acc_ref[3]) / e
        msq = m1 * m1
        cm4 = m4 - 4.0 * m1 * m3 + 6.0 * msq * m2 - 3.0 * msq * msq
        cm4 = jnp.maximum(cm4, 0.0)
        reg = -jnp.sqrt(jnp.sqrt(cm4))
        loss_ref[0, 0] = jnp.sum(acc_ref[4]) / e + 0.1 * reg


def _final(z2, e1, c2, wct, bc, et2):
    return pl.pallas_call(
        _final_body,
        grid=(EPAD // FBE,),
        in_specs=[
            pl.BlockSpec((FBE, HID), lambda i: (i, 0)),
            pl.BlockSpec((FBE, HID), lambda i: (i, 0)),
            pl.BlockSpec((HID, HID), lambda i: (0, 0)),
            pl.BlockSpec((1, HID), lambda i: (0, 0)),
            pl.BlockSpec((1, 1), lambda i: (0, 0), memory_space=pltpu.SMEM),
            pl.BlockSpec((LBLK, BE), lambda i: (i, 0)),
        ],
        out_specs=[
            pl.BlockSpec((LBLK, BE), lambda i: (i, 0)),
            pl.BlockSpec((1, 1), lambda i: (0, 0), memory_space=pltpu.SMEM),
        ],
        out_shape=[
            jax.ShapeDtypeStruct((LROWS, BE), jnp.float32),
            jax.ShapeDtypeStruct((1, 1), jnp.float32),
        ],
        scratch_shapes=[pltpu.VMEM((5, LBLK, 128), jnp.float32)],
    )(z2, e1, c2, wct, bc, et2)


def kernel(x, edge_index, edge_type,
           W_n1, b_n1, W_e1, b_e1, W_n2, b_n2, W_e2, b_e2, W_c, b_c):
    f32 = jnp.float32
    src3 = jnp.pad(edge_index[0], (0, EPAD_IDX - N_EDGES),
                   constant_values=N_NODES).reshape(TOTC_PAD, CHUNK)
    dst3 = jnp.pad(edge_index[1], (0, EPAD_IDX - N_EDGES),
                   constant_values=N_NODES).reshape(TOTC_PAD, CHUNK)
    x_pad = jnp.pad(x, ((0, NPAD - N_NODES), (0, 0)))
    w_cat = jnp.concatenate([W_n1[DIM:], W_e1[2 * HID:]], axis=1)

    p1, q = _pq_tables(x_pad, w_cat)

    zero_p = jnp.zeros((ZROWS, PW), f32)
    spd = _sc_seg_p(src3, dst3, p1, zero_p)

    wab = jnp.concatenate([W_e1[:HID], W_e1[HID:2 * HID]], axis=1)
    x1, g1, g2 = _node1(spd, p1, q, wab,
                        b_n1.reshape(1, HID), b_e1.reshape(1, HID))

    zero_h = jnp.zeros((ZROWS, HID), f32)
    e1, sed = _sc_edge1(src3, dst3, g1, g2, zero_h)

    wab2 = jnp.concatenate([W_e2[:HID], W_e2[HID:2 * HID]], axis=1)
    g3, g4 = _node2(sed, spd, x1, W_n2, wab2,
                    b_n2.reshape(1, HID), b_e2.reshape(1, HID))

    z2 = _sc_edge2(src3, dst3, g3, g4)

    et2 = jnp.pad(edge_type.astype(f32), (0, EPAD - N_EDGES)).reshape(LROWS, BE)
    pred2, loss = _final(z2, e1, W_e2[2 * HID:], W_c.reshape(1, HID),
                         b_c.reshape(1, 1), et2)
    return pred2.reshape(EPAD)[:N_EDGES], loss[0, 0]
